# Initial kernel scaffold; baseline (speedup 1.0000x reference)
import jax
import jax.numpy as jnp
from jax.experimental import pallas as pl


def kernel(input_pos_tensors, table):
    raise NotImplementedError("write your pallas kernel here")


# speedup vs baseline: 3.9603x; 3.9603x over previous
import functools
import jax
import jax.numpy as jnp
from jax import lax
from jax.experimental import pallas as pl
from jax.experimental.pallas import tpu as pltpu
from jax.experimental.pallas import tpu_sc as plsc

_NC = 2
_NS = 16
_NW = _NC * _NS


def _gather_body(chunk, nchunks, b_per_w, table_hbm, idx_hbm, out_hbm,
                 idx_v, rows_v, sem):
    wid = lax.axis_index("s") * _NC + lax.axis_index("c")
    base = wid * b_per_w

    def step(i, carry):
        off = base + i * chunk
        pltpu.sync_copy(idx_hbm.at[pl.ds(off, chunk)], idx_v)
        pltpu.async_copy(table_hbm.at[idx_v], rows_v, sem).wait()
        pltpu.sync_copy(rows_v, out_hbm.at[pl.ds(off, chunk)])
        return carry

    lax.fori_loop(0, nchunks, step, 0)


def kernel(input_pos_tensors, table):
    B0, T = input_pos_tensors.shape
    V, D = table.shape
    B = B0 * T
    idx = input_pos_tensors.reshape(B).astype(jnp.int32)

    b_per_w = B // _NW
    chunk = 512
    nchunks = b_per_w // chunk

    mesh = plsc.VectorSubcoreMesh(
        core_axis_name="c", subcore_axis_name="s",
        num_cores=_NC, num_subcores=_NS)
    run = pl.kernel(
        functools.partial(_gather_body, chunk, nchunks, b_per_w),
        out_type=jax.ShapeDtypeStruct((B, D), jnp.float32),
        mesh=mesh,
        scratch_types=[
            pltpu.VMEM((chunk,), jnp.int32),
            pltpu.VMEM((chunk, D), jnp.float32),
            pltpu.SemaphoreType.DMA,
        ],
        compiler_params=pltpu.CompilerParams(use_tc_tiling_on_sc=False),
    )
    out = run(table, idx)
    return out.reshape(B0, T, D)


# speedup vs baseline: 4.0124x; 1.0131x over previous
import functools
import jax
import jax.numpy as jnp
from jax import lax
from jax.experimental import pallas as pl
from jax.experimental.pallas import tpu as pltpu
from jax.experimental.pallas import tpu_sc as plsc

_NC = 2
_NS = 16
_NW = _NC * _NS
_NBUF = 2


def _gather_body(chunk, nchunks, b_per_w, table_hbm, idx_hbm, out_hbm,
                 idx_v, rows0, rows1, sem_i, sem_g0, sem_g1, sem_w0, sem_w1):
    wid = lax.axis_index("s") * _NC + lax.axis_index("c")
    base = wid * b_per_w
    rows = (rows0, rows1)
    sem_g = (sem_g0, sem_g1)
    sem_w = (sem_w0, sem_w1)

    pltpu.async_copy(idx_hbm.at[pl.ds(base, b_per_w)], idx_v, sem_i).wait()

    def start_gather(i, b):
        pltpu.async_copy(
            table_hbm.at[idx_v.at[pl.ds(i * chunk, chunk)]], rows[b], sem_g[b])

    def wait_gather(i, b):
        pltpu.make_async_copy(
            table_hbm.at[idx_v.at[pl.ds(i * chunk, chunk)]], rows[b],
            sem_g[b]).wait()

    def start_write(i, b):
        pltpu.async_copy(rows[b], out_hbm.at[pl.ds(base + i * chunk, chunk)],
                         sem_w[b])

    def wait_write(i, b):
        pltpu.make_async_copy(
            rows[b], out_hbm.at[pl.ds(base + i * chunk, chunk)],
            sem_w[b]).wait()

    for b in range(_NBUF):
        start_gather(b, b)

    ngroups = nchunks // _NBUF

    def group(g, carry):
        for b in range(_NBUF):
            i = g * _NBUF + b
            wait_gather(i, b)
            start_write(i, b)
            wait_write(i, b)
            start_gather(i + _NBUF, b)
        return carry

    lax.fori_loop(0, ngroups - 1, group, 0)

    for b in range(_NBUF):
        i = (ngroups - 1) * _NBUF + b
        wait_gather(i, b)
        start_write(i, b)
    for b in range(_NBUF):
        i = (ngroups - 1) * _NBUF + b
        wait_write(i, b)


def kernel(input_pos_tensors, table):
    B0, T = input_pos_tensors.shape
    V, D = table.shape
    B = B0 * T
    idx = input_pos_tensors.reshape(B).astype(jnp.int32)

    b_per_w = B // _NW
    chunk = 512
    nchunks = b_per_w // chunk

    mesh = plsc.VectorSubcoreMesh(
        core_axis_name="c", subcore_axis_name="s",
        num_cores=_NC, num_subcores=_NS)
    run = pl.kernel(
        functools.partial(_gather_body, chunk, nchunks, b_per_w),
        out_type=jax.ShapeDtypeStruct((B, D), jnp.float32),
        mesh=mesh,
        scratch_types=[
            pltpu.VMEM((b_per_w,), jnp.int32),
            pltpu.VMEM((chunk, D), jnp.float32),
            pltpu.VMEM((chunk, D), jnp.float32),
            pltpu.SemaphoreType.DMA,
            pltpu.SemaphoreType.DMA,
            pltpu.SemaphoreType.DMA,
            pltpu.SemaphoreType.DMA,
            pltpu.SemaphoreType.DMA,
        ],
        compiler_params=pltpu.CompilerParams(use_tc_tiling_on_sc=False),
    )
    out = run(table, idx)
    return out.reshape(B0, T, D)


# speedup vs baseline: 4.9942x; 1.2447x over previous
import functools
import jax
import jax.numpy as jnp
from jax import lax
from jax.experimental import pallas as pl
from jax.experimental.pallas import tpu as pltpu
from jax.experimental.pallas import tpu_sc as plsc

_NC = 2
_NS = 16
_NW = _NC * _NS
_NBUF = 2


def _gather_body(chunk, nchunks, b_per_w, table_hbm, idx_hbm, out_hbm,
                 table_sp, idx_v, rows0, rows1,
                 sem_i, sem_g0, sem_g1, sem_w0, sem_w1):
    wid = lax.axis_index("s") * _NC + lax.axis_index("c")
    base = wid * b_per_w
    rows = (rows0, rows1)
    sem_g = (sem_g0, sem_g1)
    sem_w = (sem_w0, sem_w1)

    @pl.when(lax.axis_index("s") == 0)
    def _stage():
        pltpu.sync_copy(table_hbm, table_sp)
    plsc.subcore_barrier()

    pltpu.async_copy(idx_hbm.at[pl.ds(base, b_per_w)], idx_v, sem_i).wait()

    def start_gather(i, b):
        pltpu.async_copy(
            table_sp.at[idx_v.at[pl.ds(i * chunk, chunk)]], rows[b], sem_g[b])

    def wait_gather(i, b):
        pltpu.make_async_copy(
            table_sp.at[idx_v.at[pl.ds(i * chunk, chunk)]], rows[b],
            sem_g[b]).wait()

    def start_write(i, b):
        pltpu.async_copy(rows[b], out_hbm.at[pl.ds(base + i * chunk, chunk)],
                         sem_w[b])

    def wait_write(i, b):
        pltpu.make_async_copy(
            rows[b], out_hbm.at[pl.ds(base + i * chunk, chunk)],
            sem_w[b]).wait()

    for b in range(_NBUF):
        start_gather(b, b)

    ngroups = nchunks // _NBUF

    def group(g, carry):
        for b in range(_NBUF):
            i = g * _NBUF + b
            wait_gather(i, b)
            start_write(i, b)
            wait_write(i, b)
            start_gather(i + _NBUF, b)
        return carry

    lax.fori_loop(0, ngroups - 1, group, 0)

    for b in range(_NBUF):
        i = (ngroups - 1) * _NBUF + b
        wait_gather(i, b)
        start_write(i, b)
    for b in range(_NBUF):
        i = (ngroups - 1) * _NBUF + b
        wait_write(i, b)


def kernel(input_pos_tensors, table):
    B0, T = input_pos_tensors.shape
    V, D = table.shape
    B = B0 * T
    idx = input_pos_tensors.reshape(B).astype(jnp.int32)

    b_per_w = B // _NW
    chunk = 512
    nchunks = b_per_w // chunk

    mesh = plsc.VectorSubcoreMesh(
        core_axis_name="c", subcore_axis_name="s",
        num_cores=_NC, num_subcores=_NS)
    run = pl.kernel(
        functools.partial(_gather_body, chunk, nchunks, b_per_w),
        out_type=jax.ShapeDtypeStruct((B, D), jnp.float32),
        mesh=mesh,
        scratch_types=[
            pltpu.VMEM_SHARED((V, D), jnp.float32),
            pltpu.VMEM((b_per_w,), jnp.int32),
            pltpu.VMEM((chunk, D), jnp.float32),
            pltpu.VMEM((chunk, D), jnp.float32),
            pltpu.SemaphoreType.DMA,
            pltpu.SemaphoreType.DMA,
            pltpu.SemaphoreType.DMA,
            pltpu.SemaphoreType.DMA,
            pltpu.SemaphoreType.DMA,
        ],
        compiler_params=pltpu.CompilerParams(use_tc_tiling_on_sc=False),
    )
    out = run(table, idx)
    return out.reshape(B0, T, D)


# speedup vs baseline: 5.0275x; 1.0067x over previous
import functools
import jax
import jax.numpy as jnp
from jax import lax
from jax.experimental import pallas as pl
from jax.experimental.pallas import tpu as pltpu
from jax.experimental.pallas import tpu_sc as plsc

_NC = 2
_NS = 16
_NW = _NC * _NS
_NBUF = 4


def _gather_body(chunk, nchunks, b_per_w, table_hbm, idx_hbm, out_hbm,
                 table_sp, idx_v, rows0, rows1, rows2, rows3,
                 sem_i, sem_g0, sem_g1, sem_g2, sem_g3,
                 sem_w0, sem_w1, sem_w2, sem_w3):
    wid = lax.axis_index("s") * _NC + lax.axis_index("c")
    base = wid * b_per_w
    rows = (rows0, rows1, rows2, rows3)
    sem_g = (sem_g0, sem_g1, sem_g2, sem_g3)
    sem_w = (sem_w0, sem_w1, sem_w2, sem_w3)

    @pl.when(lax.axis_index("s") == 0)
    def _stage():
        pltpu.sync_copy(table_hbm, table_sp)
    plsc.subcore_barrier()

    pltpu.async_copy(idx_hbm.at[pl.ds(base, b_per_w)], idx_v, sem_i).wait()

    def start_gather(i, b):
        pltpu.async_copy(
            table_sp.at[idx_v.at[pl.ds(i * chunk, chunk)]], rows[b], sem_g[b])

    def wait_gather(i, b):
        pltpu.make_async_copy(
            table_sp.at[idx_v.at[pl.ds(i * chunk, chunk)]], rows[b],
            sem_g[b]).wait()

    def start_write(i, b):
        pltpu.async_copy(rows[b], out_hbm.at[pl.ds(base + i * chunk, chunk)],
                         sem_w[b])

    def wait_write(i, b):
        pltpu.make_async_copy(
            rows[b], out_hbm.at[pl.ds(base + i * chunk, chunk)],
            sem_w[b]).wait()

    for b in range(_NBUF):
        start_gather(b, b)

    ngroups = nchunks // _NBUF

    def group(g, carry):
        for b in range(_NBUF):
            i = g * _NBUF + b
            wait_gather(i, b)
            start_write(i, b)
            wait_write(i, b)
            start_gather(i + _NBUF, b)
        return carry

    lax.fori_loop(0, ngroups - 1, group, 0)

    for b in range(_NBUF):
        i = (ngroups - 1) * _NBUF + b
        wait_gather(i, b)
        start_write(i, b)
    for b in range(_NBUF):
        i = (ngroups - 1) * _NBUF + b
        wait_write(i, b)


def kernel(input_pos_tensors, table):
    B0, T = input_pos_tensors.shape
    V, D = table.shape
    B = B0 * T
    idx = input_pos_tensors.reshape(B).astype(jnp.int32)

    b_per_w = B // _NW
    chunk = 256
    nchunks = b_per_w // chunk

    mesh = plsc.VectorSubcoreMesh(
        core_axis_name="c", subcore_axis_name="s",
        num_cores=_NC, num_subcores=_NS)
    run = pl.kernel(
        functools.partial(_gather_body, chunk, nchunks, b_per_w),
        out_type=jax.ShapeDtypeStruct((B, D), jnp.float32),
        mesh=mesh,
        scratch_types=[
            pltpu.VMEM_SHARED((V, D), jnp.float32),
            pltpu.VMEM((b_per_w,), jnp.int32),
            pltpu.VMEM((chunk, D), jnp.float32),
            pltpu.VMEM((chunk, D), jnp.float32),
            pltpu.VMEM((chunk, D), jnp.float32),
            pltpu.VMEM((chunk, D), jnp.float32),
            pltpu.SemaphoreType.DMA,
            pltpu.SemaphoreType.DMA,
            pltpu.SemaphoreType.DMA,
            pltpu.SemaphoreType.DMA,
            pltpu.SemaphoreType.DMA,
            pltpu.SemaphoreType.DMA,
            pltpu.SemaphoreType.DMA,
            pltpu.SemaphoreType.DMA,
            pltpu.SemaphoreType.DMA,
        ],
        compiler_params=pltpu.CompilerParams(use_tc_tiling_on_sc=False),
    )
    out = run(table, idx)
    return out.reshape(B0, T, D)
